# Initial kernel scaffold; baseline (speedup 1.0000x reference)
import jax
import jax.numpy as jnp
from jax.experimental import pallas as pl


def kernel(x, edge_index, bn0_g, bn0_b, w1, w1_b, bn1_g, bn1_b, w2, w2_b):
    raise NotImplementedError("write your pallas kernel here")


# speedup vs baseline: 3.0346x; 3.0346x over previous
import functools

import jax
import jax.numpy as jnp
from jax import lax
from jax.experimental import pallas as pl
from jax.experimental.pallas import tpu as pltpu
from jax.experimental.pallas import tpu_sc as plsc

N = 10000
E = 320000
D = 128
H = 128
C = 40
EPS = 1e-5

NC = 2
NS = 16
NW = NC * NS
CH = 128
K = 80
EPT = CH * K
EP = NW * EPT
NPAD = 10240
RPT = NPAD // NS

BLK = 1000
NB = N // BLK


def _bn0_call(x, g, b):
  def body(x_ref, g_ref, b_ref, h_ref, stats):
    p = pl.program_id(0)
    i = pl.program_id(1)

    @pl.when(p == 0)
    def _():
      xb = x_ref[...]
      s = jnp.sum(xb, axis=0, keepdims=True)
      s2 = jnp.sum(xb * xb, axis=0, keepdims=True)

      @pl.when(i == 0)
      def _():
        stats[0:1, :] = s
        stats[1:2, :] = s2

      @pl.when(i > 0)
      def _():
        stats[0:1, :] = stats[0:1, :] + s
        stats[1:2, :] = stats[1:2, :] + s2

    @pl.when(p == 1)
    def _():
      m = stats[0:1, :] / N
      v = stats[1:2, :] / N - m * m
      scale = lax.rsqrt(v + EPS) * g_ref[...]
      h_ref[...] = (x_ref[...] - m) * scale + b_ref[...]

  return pl.pallas_call(
      body,
      grid=(2, NB),
      in_specs=[
          pl.BlockSpec((BLK, D), lambda p, i: (i, 0)),
          pl.BlockSpec((1, D), lambda p, i: (0, 0)),
          pl.BlockSpec((1, D), lambda p, i: (0, 0)),
      ],
      out_specs=pl.BlockSpec((BLK, D), lambda p, i: (i * p, 0)),
      out_shape=jax.ShapeDtypeStruct((N, D), jnp.float32),
      scratch_shapes=[pltpu.VMEM((2, D), jnp.float32)],
  )(x, g.reshape(1, D), b.reshape(1, D))


def _sc_segment_sum(h, src_t, dst_t):
  mesh = plsc.VectorSubcoreMesh(core_axis_name="c", subcore_axis_name="s")

  @functools.partial(
      pl.kernel,
      out_type=jax.ShapeDtypeStruct((NC, NPAD, D), jnp.float32),
      mesh=mesh,
      scratch_types=[
          pltpu.VMEM((K, CH), jnp.int32),
          pltpu.VMEM((K, CH), jnp.int32),
          pltpu.VMEM((CH, D), jnp.float32),
          pltpu.VMEM((16, D), jnp.float32),
          pltpu.VMEM_SHARED((NPAD, D), jnp.float32),
          pltpu.SemaphoreType.DMA,
      ],
  )
  def body(h_hbm, src_hbm, dst_hbm, out_hbm, sidx, didx, rows, zbuf, acc, sem):
    c = lax.axis_index("c")
    s = lax.axis_index("s")
    wid = s * NC + c
    rbase = s * RPT

    zeros16 = jnp.zeros((16,), jnp.float32)
    for i in range(16):
      for j in range(D // 16):
        zbuf[i, pl.ds(j * 16, 16)] = zeros16

    def zloop(t, carry):
      pltpu.sync_copy(zbuf, acc.at[pl.ds(rbase + t * 16, 16), :])
      return carry

    lax.fori_loop(0, RPT // 16, zloop, 0)

    pltpu.sync_copy(src_hbm.at[wid], sidx)
    pltpu.sync_copy(dst_hbm.at[wid], didx)

    plsc.subcore_barrier()

    def chunk(j, carry):
      pltpu.async_copy(h_hbm.at[sidx.at[j]], rows, sem).wait()
      pltpu.sync_copy(rows, acc.at[didx.at[j]], add=True)
      return carry

    lax.fori_loop(0, K, chunk, 0)

    plsc.subcore_barrier()

    pltpu.sync_copy(acc.at[pl.ds(rbase, RPT), :],
                    out_hbm.at[c, pl.ds(rbase, RPT), :])

  return body(h, src_t, dst_t)


def _mlp_call(h, parts, w1, b1, g1, be1, w2, b2):
  def body(h_ref, parts_ref, w1_ref, b1_ref, g1_ref, be1_ref, w2_ref, b2_ref,
           out_ref, u_s, stats):
    p = pl.program_id(0)
    i = pl.program_id(1)

    @pl.when(p == 0)
    def _():
      z = h_ref[...] + parts_ref[0] + parts_ref[1]
      u = jnp.dot(z, w1_ref[...], preferred_element_type=jnp.float32)
      u = jnp.maximum(u + b1_ref[...], 0.0)
      u_s[i] = u
      s = jnp.sum(u, axis=0, keepdims=True)
      s2 = jnp.sum(u * u, axis=0, keepdims=True)

      @pl.when(i == 0)
      def _():
        stats[0:1, :] = s
        stats[1:2, :] = s2

      @pl.when(i > 0)
      def _():
        stats[0:1, :] = stats[0:1, :] + s
        stats[1:2, :] = stats[1:2, :] + s2

    @pl.when(p == 1)
    def _():
      m = stats[0:1, :] / N
      v = stats[1:2, :] / N - m * m
      u = u_s[i]
      un = (u - m) * lax.rsqrt(v + EPS) * g1_ref[...] + be1_ref[...]
      out_ref[...] = jnp.dot(un, w2_ref[...],
                             preferred_element_type=jnp.float32) + b2_ref[...]

  return pl.pallas_call(
      body,
      grid=(2, NB),
      in_specs=[
          pl.BlockSpec((BLK, D), lambda p, i: (i * (1 - p), 0)),
          pl.BlockSpec((NC, BLK, D), lambda p, i: (0, i * (1 - p), 0)),
          pl.BlockSpec((D, H), lambda p, i: (0, 0)),
          pl.BlockSpec((1, H), lambda p, i: (0, 0)),
          pl.BlockSpec((1, H), lambda p, i: (0, 0)),
          pl.BlockSpec((1, H), lambda p, i: (0, 0)),
          pl.BlockSpec((H, C), lambda p, i: (0, 0)),
          pl.BlockSpec((1, C), lambda p, i: (0, 0)),
      ],
      out_specs=pl.BlockSpec((BLK, C), lambda p, i: (i * p, 0)),
      out_shape=jax.ShapeDtypeStruct((N, C), jnp.float32),
      scratch_shapes=[
          pltpu.VMEM((NB, BLK, H), jnp.float32),
          pltpu.VMEM((2, H), jnp.float32),
      ],
  )(h, parts, w1, b1.reshape(1, H), g1.reshape(1, H), be1.reshape(1, H),
    w2, b2.reshape(1, C))


def kernel(x, edge_index, bn0_g, bn0_b, w1, w1_b, bn1_g, bn1_b, w2, w2_b):
  h = _bn0_call(x, bn0_g, bn0_b)

  pad = EP - E
  src_t = jnp.concatenate(
      [edge_index[0], jnp.zeros((pad,), jnp.int32)]).reshape(NW, K, CH)
  dst_t = jnp.concatenate(
      [edge_index[1], jnp.full((pad,), N, jnp.int32)]).reshape(NW, K, CH)

  parts = _sc_segment_sum(h, src_t, dst_t)

  return _mlp_call(h, parts, w1, w1_b, bn1_g, bn1_b, w2, w2_b)


# speedup vs baseline: 10.0578x; 3.3144x over previous
import functools

import jax
import jax.numpy as jnp
from jax import lax
from jax.experimental import pallas as pl
from jax.experimental.pallas import tpu as pltpu
from jax.experimental.pallas import tpu_sc as plsc

N = 10000
E = 320000
D = 128
H = 128
C = 40
EPS = 1e-5

NC = 2
NS = 16
NW = NC * NS
CH = 128
K = 80
EPT = CH * K
EP = NW * EPT
NPAD = 10240
RPT = NPAD // NS

BLK = 1000
NB = N // BLK


def _bn0_call(x, g, b):
  def body(x_ref, g_ref, b_ref, h_ref, stats):
    p = pl.program_id(0)
    i = pl.program_id(1)

    @pl.when(p == 0)
    def _():
      xb = x_ref[...]
      s = jnp.sum(xb, axis=0, keepdims=True)
      s2 = jnp.sum(xb * xb, axis=0, keepdims=True)

      @pl.when(i == 0)
      def _():
        stats[0:1, :] = s
        stats[1:2, :] = s2

      @pl.when(i > 0)
      def _():
        stats[0:1, :] = stats[0:1, :] + s
        stats[1:2, :] = stats[1:2, :] + s2

    @pl.when(p == 1)
    def _():
      m = stats[0:1, :] / N
      v = stats[1:2, :] / N - m * m
      scale = lax.rsqrt(v + EPS) * g_ref[...]
      h_ref[...] = (x_ref[...] - m) * scale + b_ref[...]

  return pl.pallas_call(
      body,
      grid=(2, NB),
      in_specs=[
          pl.BlockSpec((BLK, D), lambda p, i: (i, 0)),
          pl.BlockSpec((1, D), lambda p, i: (0, 0)),
          pl.BlockSpec((1, D), lambda p, i: (0, 0)),
      ],
      out_specs=pl.BlockSpec((BLK, D), lambda p, i: (i * p, 0)),
      out_shape=jax.ShapeDtypeStruct((N, D), jnp.float32),
      scratch_shapes=[pltpu.VMEM((2, D), jnp.float32)],
  )(x, g.reshape(1, D), b.reshape(1, D))


NBUF = 2


def _sc_segment_sum(h, src_t, dst_t):
  mesh = plsc.VectorSubcoreMesh(core_axis_name="c", subcore_axis_name="s")

  KH = K // 2

  @functools.partial(
      pl.kernel,
      out_type=jax.ShapeDtypeStruct((NC, NPAD, D), jnp.float32),
      mesh=mesh,
      scratch_types=[
          pltpu.VMEM((KH, CH), jnp.int32),
          pltpu.VMEM((KH, CH), jnp.int32),
          [pltpu.VMEM((CH, D), jnp.float32) for _ in range(NBUF)],
          pltpu.VMEM((16, D), jnp.float32),
          pltpu.VMEM_SHARED((NPAD, D), jnp.float32),
          [pltpu.SemaphoreType.DMA for _ in range(NBUF)],
          pltpu.SemaphoreType.DMA,
      ],
  )
  def body(h_hbm, src_hbm, dst_hbm, out_hbm, sidx, didx, rows, zbuf, acc,
           gsem, ssem):
    c = lax.axis_index("c")
    s = lax.axis_index("s")
    wid = s * NC + c
    rbase = s * RPT

    zeros16 = jnp.zeros((16,), jnp.float32)
    for i in range(16):
      for j in range(D // 16):
        zbuf[i, pl.ds(j * 16, 16)] = zeros16

    def zloop(t, carry):
      pltpu.sync_copy(zbuf, acc.at[pl.ds(rbase + t * 16, 16), :])
      return carry

    lax.fori_loop(0, RPT // 16, zloop, 0)

    plsc.subcore_barrier()

    for half in range(2):
      pltpu.sync_copy(src_hbm.at[wid, pl.ds(half * KH, KH)], sidx)
      pltpu.sync_copy(dst_hbm.at[wid, pl.ds(half * KH, KH)], didx)

      for b in range(NBUF):
        pltpu.async_copy(h_hbm.at[sidx.at[b]], rows[b], gsem[b])

      def group(g, carry):
        for b in range(NBUF):
          j = g * NBUF + b
          pltpu.make_async_copy(h_hbm.at[sidx.at[j]], rows[b], gsem[b]).wait()
          pltpu.async_copy(rows[b], acc.at[didx.at[j]], ssem, add=True)
          pltpu.make_async_copy(rows[b], acc.at[didx.at[j]], ssem).wait()

          @pl.when(j < KH - NBUF)
          def _():
            pltpu.async_copy(h_hbm.at[sidx.at[j + NBUF]], rows[b], gsem[b])

        return carry

      lax.fori_loop(0, KH // NBUF, group, 0)

    plsc.subcore_barrier()

    pltpu.sync_copy(acc.at[pl.ds(rbase, RPT), :],
                    out_hbm.at[c, pl.ds(rbase, RPT), :])

  return body(h, src_t, dst_t)


def _mlp_call(h, parts, w1, b1, g1, be1, w2, b2):
  def body(h_ref, parts_ref, w1_ref, b1_ref, g1_ref, be1_ref, w2_ref, b2_ref,
           out_ref, u_s, stats):
    p = pl.program_id(0)
    i = pl.program_id(1)

    @pl.when(p == 0)
    def _():
      z = h_ref[...] + parts_ref[0] + parts_ref[1]
      u = jnp.dot(z, w1_ref[...], preferred_element_type=jnp.float32)
      u = jnp.maximum(u + b1_ref[...], 0.0)
      u_s[i] = u
      s = jnp.sum(u, axis=0, keepdims=True)
      s2 = jnp.sum(u * u, axis=0, keepdims=True)

      @pl.when(i == 0)
      def _():
        stats[0:1, :] = s
        stats[1:2, :] = s2

      @pl.when(i > 0)
      def _():
        stats[0:1, :] = stats[0:1, :] + s
        stats[1:2, :] = stats[1:2, :] + s2

    @pl.when(p == 1)
    def _():
      m = stats[0:1, :] / N
      v = stats[1:2, :] / N - m * m
      u = u_s[i]
      un = (u - m) * lax.rsqrt(v + EPS) * g1_ref[...] + be1_ref[...]
      out_ref[...] = jnp.dot(un, w2_ref[...],
                             preferred_element_type=jnp.float32) + b2_ref[...]

  return pl.pallas_call(
      body,
      grid=(2, NB),
      in_specs=[
          pl.BlockSpec((BLK, D), lambda p, i: (i * (1 - p), 0)),
          pl.BlockSpec((NC, BLK, D), lambda p, i: (0, i * (1 - p), 0)),
          pl.BlockSpec((D, H), lambda p, i: (0, 0)),
          pl.BlockSpec((1, H), lambda p, i: (0, 0)),
          pl.BlockSpec((1, H), lambda p, i: (0, 0)),
          pl.BlockSpec((1, H), lambda p, i: (0, 0)),
          pl.BlockSpec((H, C), lambda p, i: (0, 0)),
          pl.BlockSpec((1, C), lambda p, i: (0, 0)),
      ],
      out_specs=pl.BlockSpec((BLK, C), lambda p, i: (i * p, 0)),
      out_shape=jax.ShapeDtypeStruct((N, C), jnp.float32),
      scratch_shapes=[
          pltpu.VMEM((NB, BLK, H), jnp.float32),
          pltpu.VMEM((2, H), jnp.float32),
      ],
  )(h, parts, w1, b1.reshape(1, H), g1.reshape(1, H), be1.reshape(1, H),
    w2, b2.reshape(1, C))


def kernel(x, edge_index, bn0_g, bn0_b, w1, w1_b, bn1_g, bn1_b, w2, w2_b):
  h = _bn0_call(x, bn0_g, bn0_b)

  pad = EP - E
  pad_ar = jnp.arange(pad, dtype=jnp.int32)
  src_t = jnp.concatenate(
      [edge_index[0], pad_ar % N]).reshape(NW, K, CH)
  dst_t = jnp.concatenate(
      [edge_index[1], N + pad_ar % (NPAD - N)]).reshape(NW, K, CH)

  parts = _sc_segment_sum(h, src_t, dst_t)

  return _mlp_call(h, parts, w1, w1_b, bn1_g, bn1_b, w2, w2_b)


# speedup vs baseline: 10.0647x; 1.0007x over previous
import functools

import jax
import jax.numpy as jnp
from jax import lax
from jax.experimental import pallas as pl
from jax.experimental.pallas import tpu as pltpu
from jax.experimental.pallas import tpu_sc as plsc

N = 10000
E = 320000
D = 128
H = 128
C = 40
EPS = 1e-5

NC = 2
NS = 16
NW = NC * NS
CH = 128
K = 80
EPT = CH * K
EP = NW * EPT
NPAD = 10240
RPT = NPAD // NS

BLK = 1000
NB = N // BLK


def _bn0_call(x, g, b):
  def body(x_ref, g_ref, b_ref, h_ref, stats):
    p = pl.program_id(0)
    i = pl.program_id(1)

    @pl.when(p == 0)
    def _():
      xb = x_ref[...]
      s = jnp.sum(xb, axis=0, keepdims=True)
      s2 = jnp.sum(xb * xb, axis=0, keepdims=True)

      @pl.when(i == 0)
      def _():
        stats[0:1, :] = s
        stats[1:2, :] = s2

      @pl.when(i > 0)
      def _():
        stats[0:1, :] = stats[0:1, :] + s
        stats[1:2, :] = stats[1:2, :] + s2

    @pl.when(p == 1)
    def _():
      m = stats[0:1, :] / N
      v = stats[1:2, :] / N - m * m
      scale = lax.rsqrt(v + EPS) * g_ref[...]
      h_ref[...] = (x_ref[...] - m) * scale + b_ref[...]

  return pl.pallas_call(
      body,
      grid=(2, NB),
      in_specs=[
          pl.BlockSpec((BLK, D), lambda p, i: (i, 0)),
          pl.BlockSpec((1, D), lambda p, i: (0, 0)),
          pl.BlockSpec((1, D), lambda p, i: (0, 0)),
      ],
      out_specs=pl.BlockSpec((BLK, D), lambda p, i: (i * p, 0)),
      out_shape=jax.ShapeDtypeStruct((N, D), jnp.float32),
      scratch_shapes=[pltpu.VMEM((2, D), jnp.float32)],
  )(x, g.reshape(1, D), b.reshape(1, D))


NBUF = 2


def _sc_segment_sum(h, src_t, dst_t):
  mesh = plsc.VectorSubcoreMesh(core_axis_name="c", subcore_axis_name="s")

  KH = K // 2

  @functools.partial(
      pl.kernel,
      out_type=jax.ShapeDtypeStruct((NC, NPAD, D), jnp.float32),
      mesh=mesh,
      scratch_types=[
          pltpu.VMEM((KH, CH), jnp.int32),
          pltpu.VMEM((KH, CH), jnp.int32),
          [pltpu.VMEM((CH, D), jnp.float32) for _ in range(NBUF)],
          pltpu.VMEM((16, D), jnp.float32),
          pltpu.VMEM_SHARED((NPAD, D), jnp.float32),
          [pltpu.SemaphoreType.DMA for _ in range(NBUF)],
          pltpu.SemaphoreType.DMA,
      ],
  )
  def body(h_hbm, src_hbm, dst_hbm, out_hbm, sidx, didx, rows, zbuf, acc,
           gsem, ssem):
    c = lax.axis_index("c")
    s = lax.axis_index("s")
    wid = s * NC + c
    rbase = s * RPT

    zeros16 = jnp.zeros((16,), jnp.float32)
    for i in range(16):
      for j in range(D // 16):
        zbuf[i, pl.ds(j * 16, 16)] = zeros16

    def zloop(t, carry):
      pltpu.sync_copy(zbuf, acc.at[pl.ds(rbase + t * 16, 16), :])
      return carry

    lax.fori_loop(0, RPT // 16, zloop, 0)

    plsc.subcore_barrier()

    for half in range(2):
      pltpu.sync_copy(src_hbm.at[wid, pl.ds(half * KH, KH)], sidx)
      pltpu.sync_copy(dst_hbm.at[wid, pl.ds(half * KH, KH)], didx)

      pltpu.async_copy(h_hbm.at[sidx.at[0]], rows[0], gsem[0])

      def group(g, carry):
        for b in range(NBUF):
          j = g * NBUF + b
          o = 1 - b

          @pl.when(j > 0)
          def _():
            pltpu.make_async_copy(rows[o], acc.at[didx.at[j]], ssem).wait()

          @pl.when(j + 1 < KH)
          def _():
            pltpu.async_copy(h_hbm.at[sidx.at[j + 1]], rows[o], gsem[o])

          pltpu.make_async_copy(h_hbm.at[sidx.at[j]], rows[b], gsem[b]).wait()
          pltpu.async_copy(rows[b], acc.at[didx.at[j]], ssem, add=True)

        return carry

      lax.fori_loop(0, KH // NBUF, group, 0)
      pltpu.make_async_copy(rows[(KH - 1) % 2], acc.at[didx.at[KH - 1]],
                            ssem).wait()

    plsc.subcore_barrier()

    pltpu.sync_copy(acc.at[pl.ds(rbase, RPT), :],
                    out_hbm.at[c, pl.ds(rbase, RPT), :])

  return body(h, src_t, dst_t)


def _mlp_call(h, parts, w1, b1, g1, be1, w2, b2):
  def body(h_ref, parts_ref, w1_ref, b1_ref, g1_ref, be1_ref, w2_ref, b2_ref,
           out_ref, u_s, stats):
    p = pl.program_id(0)
    i = pl.program_id(1)

    @pl.when(p == 0)
    def _():
      z = h_ref[...] + parts_ref[0] + parts_ref[1]
      u = jnp.dot(z, w1_ref[...], preferred_element_type=jnp.float32)
      u = jnp.maximum(u + b1_ref[...], 0.0)
      u_s[i] = u
      s = jnp.sum(u, axis=0, keepdims=True)
      s2 = jnp.sum(u * u, axis=0, keepdims=True)

      @pl.when(i == 0)
      def _():
        stats[0:1, :] = s
        stats[1:2, :] = s2

      @pl.when(i > 0)
      def _():
        stats[0:1, :] = stats[0:1, :] + s
        stats[1:2, :] = stats[1:2, :] + s2

    @pl.when(p == 1)
    def _():
      m = stats[0:1, :] / N
      v = stats[1:2, :] / N - m * m
      u = u_s[i]
      un = (u - m) * lax.rsqrt(v + EPS) * g1_ref[...] + be1_ref[...]
      out_ref[...] = jnp.dot(un, w2_ref[...],
                             preferred_element_type=jnp.float32) + b2_ref[...]

  return pl.pallas_call(
      body,
      grid=(2, NB),
      in_specs=[
          pl.BlockSpec((BLK, D), lambda p, i: (i * (1 - p), 0)),
          pl.BlockSpec((NC, BLK, D), lambda p, i: (0, i * (1 - p), 0)),
          pl.BlockSpec((D, H), lambda p, i: (0, 0)),
          pl.BlockSpec((1, H), lambda p, i: (0, 0)),
          pl.BlockSpec((1, H), lambda p, i: (0, 0)),
          pl.BlockSpec((1, H), lambda p, i: (0, 0)),
          pl.BlockSpec((H, C), lambda p, i: (0, 0)),
          pl.BlockSpec((1, C), lambda p, i: (0, 0)),
      ],
      out_specs=pl.BlockSpec((BLK, C), lambda p, i: (i * p, 0)),
      out_shape=jax.ShapeDtypeStruct((N, C), jnp.float32),
      scratch_shapes=[
          pltpu.VMEM((NB, BLK, H), jnp.float32),
          pltpu.VMEM((2, H), jnp.float32),
      ],
  )(h, parts, w1, b1.reshape(1, H), g1.reshape(1, H), be1.reshape(1, H),
    w2, b2.reshape(1, C))


def kernel(x, edge_index, bn0_g, bn0_b, w1, w1_b, bn1_g, bn1_b, w2, w2_b):
  h = _bn0_call(x, bn0_g, bn0_b)

  pad = EP - E
  pad_ar = jnp.arange(pad, dtype=jnp.int32)
  src_t = jnp.concatenate(
      [edge_index[0], pad_ar % N]).reshape(NW, K, CH)
  dst_t = jnp.concatenate(
      [edge_index[1], N + pad_ar % (NPAD - N)]).reshape(NW, K, CH)

  parts = _sc_segment_sum(h, src_t, dst_t)

  return _mlp_call(h, parts, w1, w1_b, bn1_g, bn1_b, w2, w2_b)
